# baseline (speedup 1.0000x reference)
import functools

import jax
import jax.numpy as jnp
from jax import lax
from jax.experimental import pallas as pl
from jax.experimental.pallas import tpu as pltpu
from jax.experimental.pallas import tpu_sc as plsc

B, S, D, V = 4, 2048, 768, 100000
NC, NS = 2, 16
NW = NC * NS
CHUNK = S // NW
W = 32
NHALF = CHUNK // W
NSUB = B * NHALF
NBUF = 3
LANES = 16


def _build():
    mesh = plsc.VectorSubcoreMesh(core_axis_name="c", subcore_axis_name="s")

    @functools.partial(
        pl.kernel,
        mesh=mesh,
        out_type=jax.ShapeDtypeStruct((B, S, D), jnp.float32),
        scratch_types=[
            pltpu.VMEM((B, CHUNK), jnp.int32),
            pltpu.VMEM((CHUNK, D), jnp.float32),
            pltpu.VMEM((W, D), jnp.float32),
            pltpu.VMEM((W, D), jnp.float32),
            pltpu.VMEM((W, D), jnp.float32),
            pltpu.SemaphoreType.DMA,
            pltpu.SemaphoreType.DMA,
        ],
    )
    def emb_kernel(x_hbm, tok_hbm, pos_hbm, out_hbm,
                   idx_v, pos_v, buf0, buf1, buf2, gsem, ssem):
        wid = lax.axis_index("s") * NC + lax.axis_index("c")
        base = wid * CHUNK
        bufs = (buf0, buf1, buf2)

        pltpu.sync_copy(pos_hbm.at[pl.ds(base, CHUNK)], pos_v)
        for b in range(B):
            pltpu.sync_copy(x_hbm.at[b, pl.ds(base, CHUNK)], idx_v.at[b])

        def gather(k):
            b, h = divmod(k, NHALF)
            return pltpu.async_copy(
                tok_hbm.at[idx_v.at[b, pl.ds(h * W, W)]], bufs[k % NBUF], gsem)

        def store(k):
            b, h = divmod(k, NHALF)
            return pltpu.async_copy(
                bufs[k % NBUF], out_hbm.at[b, pl.ds(base + h * W, W)], ssem)

        gathers = [None] * NSUB
        stores = [None] * NSUB
        gathers[0] = gather(0)
        for k in range(NSUB):
            if k + 1 < NSUB:
                if k + 1 >= NBUF:
                    stores[k + 1 - NBUF].wait()
                gathers[k + 1] = gather(k + 1)
            gathers[k].wait()

            h = k % NHALF
            buf = bufs[k % NBUF]

            def add_row(r, _):
                for j in range(D // LANES):
                    sl = pl.ds(j * LANES, LANES)
                    buf[r, sl] = buf[r, sl] + pos_v[h * W + r, sl]
                return 0

            lax.fori_loop(0, W, add_row, 0)
            stores[k] = store(k)
        for k in range(NSUB - NBUF, NSUB):
            stores[k].wait()

    return emb_kernel


_emb = _build()


def kernel(x, token_emb, pos_emb):
    return _emb(x.astype(jnp.int32), token_emb, pos_emb)


# speedup vs baseline: 1.1090x; 1.1090x over previous
import functools

import jax
import jax.numpy as jnp
from jax import lax
from jax.experimental import pallas as pl
from jax.experimental.pallas import tpu as pltpu
from jax.experimental.pallas import tpu_sc as plsc

B, S, D, V = 4, 2048, 768, 100000
NC, NS = 2, 16
NW = NC * NS
CHUNK = S // NW
W = 32
NHALF = CHUNK // W
NSUB = B * NHALF
NBUF = 3
LANES = 16


def _build():
    mesh = plsc.VectorSubcoreMesh(core_axis_name="c", subcore_axis_name="s")

    @functools.partial(
        pl.kernel,
        mesh=mesh,
        out_type=jax.ShapeDtypeStruct((B, S, D), jnp.float32),
        scratch_types=[
            pltpu.VMEM((B, CHUNK), jnp.int32),
            pltpu.VMEM((CHUNK, D), jnp.float32),
            pltpu.VMEM((W, D), jnp.float32),
            pltpu.VMEM((W, D), jnp.float32),
            pltpu.VMEM((W, D), jnp.float32),
            pltpu.SemaphoreType.DMA,
            pltpu.SemaphoreType.DMA,
            pltpu.SemaphoreType.DMA,
        ],
    )
    def emb_kernel(x_hbm, tok_hbm, pos_hbm, out_hbm,
                   idx_v, pos_v, buf0, buf1, buf2, gsem, ssem, psem):
        wid = lax.axis_index("s") * NC + lax.axis_index("c")
        base = wid * CHUNK
        bufs = (buf0, buf1, buf2)

        pos_cp = pltpu.async_copy(pos_hbm.at[pl.ds(base, CHUNK)], pos_v, psem)
        for b in range(B):
            pltpu.sync_copy(x_hbm.at[b, pl.ds(base, CHUNK)], idx_v.at[b])

        def gather(k):
            b, h = divmod(k, NHALF)
            return pltpu.async_copy(
                tok_hbm.at[idx_v.at[b, pl.ds(h * W, W)]], bufs[k % NBUF], gsem)

        def store(k):
            b, h = divmod(k, NHALF)
            return pltpu.async_copy(
                bufs[k % NBUF], out_hbm.at[b, pl.ds(base + h * W, W)], ssem)

        gathers = [None] * NSUB
        stores = [None] * NSUB
        gathers[0] = gather(0)
        for k in range(NSUB):
            if k + 1 < NSUB:
                if k + 1 >= NBUF:
                    stores[k + 1 - NBUF].wait()
                gathers[k + 1] = gather(k + 1)
            gathers[k].wait()
            if k == 0:
                pos_cp.wait()

            h = k % NHALF
            buf = bufs[k % NBUF]

            def add_row(r, _):
                for j in range(D // LANES):
                    sl = pl.ds(j * LANES, LANES)
                    plsc.addupdate(buf.at[r, sl], pos_v[h * W + r, sl])
                return 0

            lax.fori_loop(0, W, add_row, 0)
            stores[k] = store(k)
        for k in range(NSUB - NBUF, NSUB):
            stores[k].wait()

    return emb_kernel


_emb = _build()


def kernel(x, token_emb, pos_emb):
    return _emb(x.astype(jnp.int32), token_emb, pos_emb)
